# baseline (speedup 1.0000x reference)
import jax
import jax.numpy as jnp
from jax.experimental import pallas as pl
from jax.experimental.pallas import tpu as pltpu

HIDDEN = 2048
N_EXPERTS = 64
BLOCK_M = 1024


def _router_kernel(hs_a, hs_b, w_ref, out_ref):
    w = w_ref[...]
    dims = (((1,), (1,)), ((), ()))
    out_ref[:BLOCK_M, :] = jax.lax.dot_general(
        hs_a[...], w, dimension_numbers=dims,
        preferred_element_type=jnp.float32)
    out_ref[BLOCK_M:, :] = jax.lax.dot_general(
        hs_b[...], w, dimension_numbers=dims,
        preferred_element_type=jnp.float32)


def kernel(hidden_states, weight):
    hs = hidden_states.reshape(-1, HIDDEN).astype(jnp.float32)
    w = weight.astype(jnp.float32)
    m = hs.shape[0]
    return pl.pallas_call(
        _router_kernel,
        grid=(m // (2 * BLOCK_M),),
        in_specs=[
            pl.BlockSpec((BLOCK_M, HIDDEN), lambda i: (2 * i, 0)),
            pl.BlockSpec((BLOCK_M, HIDDEN), lambda i: (2 * i + 1, 0)),
            pl.BlockSpec(memory_space=pltpu.VMEM),
        ],
        out_specs=pl.BlockSpec((2 * BLOCK_M, N_EXPERTS), lambda i: (i, 0)),
        out_shape=jax.ShapeDtypeStruct((m, N_EXPERTS), jnp.float32),
    )(hs, hs, w)


# speedup vs baseline: 1.0009x; 1.0009x over previous
import jax
import jax.numpy as jnp
from jax.experimental import pallas as pl
from jax.experimental.pallas import tpu as pltpu

HIDDEN = 2048
N_EXPERTS = 64
BLOCK_M = 1024


def _router_kernel(hs_a, hs_b, w_ref, out_ref):
    w = w_ref[...]
    dims = (((1,), (1,)), ((), ()))
    out_ref[0] = jax.lax.dot_general(
        hs_a[0], w, dimension_numbers=dims,
        preferred_element_type=jnp.float32)
    out_ref[1] = jax.lax.dot_general(
        hs_b[0], w, dimension_numbers=dims,
        preferred_element_type=jnp.float32)


def kernel(hidden_states, weight):
    hs = hidden_states.reshape(-1, HIDDEN).astype(jnp.float32)
    w = weight.astype(jnp.float32)
    m = hs.shape[0]
    half = m // 2
    hs2 = hs.reshape(2, half, HIDDEN)
    out = pl.pallas_call(
        _router_kernel,
        grid=(half // BLOCK_M,),
        in_specs=[
            pl.BlockSpec((1, BLOCK_M, HIDDEN), lambda i: (0, i, 0)),
            pl.BlockSpec((1, BLOCK_M, HIDDEN), lambda i: (1, i, 0)),
            pl.BlockSpec(memory_space=pltpu.VMEM),
        ],
        out_specs=pl.BlockSpec((2, BLOCK_M, N_EXPERTS), lambda i: (0, i, 0)),
        out_shape=jax.ShapeDtypeStruct((2, half, N_EXPERTS), jnp.float32),
    )(hs2, hs2, w)
    return out.reshape(m, N_EXPERTS)
